# baseline (speedup 1.0000x reference)
import functools

import jax
import jax.numpy as jnp
from jax import lax
from jax.experimental import pallas as pl
from jax.experimental.pallas import tpu as pltpu

_NUM_SEM = 16
_NSEM1 = _NUM_SEM + 1
_ROWS = 4


def _split(a):
    hi = a.astype(jnp.bfloat16)
    lo = (a - hi.astype(jnp.float32)).astype(jnp.bfloat16)
    return hi, lo


def _mm(a, b):
    return lax.dot_general(a, b, (((1,), (0,)), ((), ())),
                           preferred_element_type=jnp.float32)


def _dot3(a, b):
    ahi, alo = _split(a)
    bhi, blo = _split(b)
    return _mm(ahi, bhi) + (_mm(ahi, blo) + _mm(alo, bhi))


def _dot2(a, b_exact_bf16):
    ahi, alo = _split(a)
    return _mm(ahi, b_exact_bf16) + _mm(alo, b_exact_bf16)


def _body(params_ref, depth_ref, sem_ref, embt_ref, w1at_ref, w1bt_ref,
          b1_ref, w2t_ref, b2_ref, w3t_ref, b3_ref,
          pts_ref, lbl_ref, feat_ref, *, width, rows):
    i = pl.program_id(0)
    d = depth_ref[0]
    lab = sem_ref[0]
    bsz = d.shape[1]

    valid = (d > 0.1) & (d < 10.0)
    lbl = jnp.where(valid, lab, 0)
    lbl_ref[0] = lbl

    pidx = lax.broadcasted_iota(jnp.int32, (1, bsz), 1)
    colf = (pidx % width).astype(jnp.float32)
    rowf = (i * rows + pidx // width).astype(jnp.float32)
    cx = params_ref[0]
    cy = params_ref[1]
    rfx = params_ref[2]
    rfy = params_ref[3]
    zero = jnp.zeros_like(d)
    x = jnp.where(valid, (colf - cx) * d * rfx, zero)
    y = jnp.where(valid, (rowf - cy) * d * rfy, zero)
    z = jnp.where(valid, d, zero)
    pts_t = jnp.concatenate([x, y, z], axis=0)
    pts_ref[...] = pts_t

    lc = jnp.clip(lbl, 0, _NUM_SEM)
    sem_iota = lax.broadcasted_iota(jnp.int32, (_NSEM1, bsz), 0)
    one_t = jnp.where(sem_iota == lc, 1.0, 0.0).astype(jnp.bfloat16)

    t1t = lax.dot_general(w1bt_ref[...], embt_ref[...],
                          (((1,), (0,)), ((), ())),
                          precision=lax.Precision.HIGHEST,
                          preferred_element_type=jnp.float32)
    h1 = _dot3(w1at_ref[...], pts_t) + _dot2(t1t, one_t) + b1_ref[...]
    h1 = jnp.maximum(h1, 0.0)
    h2 = jnp.maximum(_dot3(w2t_ref[...], h1) + b2_ref[...], 0.0)
    feat_ref[...] = _dot3(w3t_ref[...], h2) + b3_ref[...]


def kernel(semantic_map, depth, intrinsics, emb_table, w1, b1, w2, b2, w3, b3):
    h, w = depth.shape
    n = h * w
    rows = _ROWS
    bsz = rows * w
    grid = h // rows
    in_dim = 3 + emb_table.shape[1]

    fx, fy = intrinsics[0, 0], intrinsics[1, 1]
    cx, cy = intrinsics[0, 2], intrinsics[1, 2]
    params = jnp.stack([cx, cy, 1.0 / fx, 1.0 / fy]).astype(jnp.float32)

    depth3 = depth.reshape(grid, 1, bsz)
    sem3 = semantic_map.astype(jnp.int32).reshape(grid, 1, bsz)
    embt = emb_table.T
    w1at = w1[:3, :].T
    w1bt = w1[3:, :].T
    w2t = w2.T
    w3t = w3.T
    b1c = b1.reshape(-1, 1)
    b2c = b2.reshape(-1, 1)
    b3c = b3.reshape(-1, 1)

    body = functools.partial(_body, width=w, rows=rows)

    const = lambda *shape: pl.BlockSpec(shape, lambda i: (0,) * len(shape))
    pts_t, lbl3, feat_t = pl.pallas_call(
        body,
        grid=(grid,),
        in_specs=[
            pl.BlockSpec(memory_space=pltpu.SMEM),
            pl.BlockSpec((1, 1, bsz), lambda i: (i, 0, 0)),
            pl.BlockSpec((1, 1, bsz), lambda i: (i, 0, 0)),
            const(*embt.shape),
            const(*w1at.shape),
            const(*w1bt.shape),
            const(*b1c.shape),
            const(*w2t.shape),
            const(*b2c.shape),
            const(*w3t.shape),
            const(*b3c.shape),
        ],
        out_specs=[
            pl.BlockSpec((3, bsz), lambda i: (0, i)),
            pl.BlockSpec((1, 1, bsz), lambda i: (i, 0, 0)),
            pl.BlockSpec((in_dim, bsz), lambda i: (0, i)),
        ],
        out_shape=[
            jax.ShapeDtypeStruct((3, n), jnp.float32),
            jax.ShapeDtypeStruct((grid, 1, bsz), jnp.int32),
            jax.ShapeDtypeStruct((in_dim, n), jnp.float32),
        ],
        compiler_params=pltpu.CompilerParams(
            dimension_semantics=("arbitrary",)),
    )(params, depth3, sem3, embt, w1at, w1bt, b1c, w2t, b2c, w3t, b3c)

    return pts_t.T, lbl3.reshape(n), feat_t.T


# speedup vs baseline: 2.3159x; 2.3159x over previous
import functools

import jax
import jax.numpy as jnp
from jax import lax
from jax.experimental import pallas as pl
from jax.experimental.pallas import tpu as pltpu

_NUM_SEM = 16
_NSEM1 = _NUM_SEM + 1
_ROWS = 8


def _split(a):
    hi = a.astype(jnp.bfloat16)
    lo = (a - hi.astype(jnp.float32)).astype(jnp.bfloat16)
    return hi, lo


def _mm(a, b):
    return lax.dot_general(a, b, (((1,), (0,)), ((), ())),
                           preferred_element_type=jnp.float32)


def _dot3(a, b):
    ahi, alo = _split(a)
    bhi, blo = _split(b)
    return _mm(ahi, bhi) + (_mm(ahi, blo) + _mm(alo, bhi))


def _dot2(a, b_exact_bf16):
    ahi, alo = _split(a)
    return _mm(ahi, b_exact_bf16) + _mm(alo, b_exact_bf16)


def _body(params_ref, depth_ref, sem_ref, embt_ref, w1at_ref, w1bt_ref,
          b1_ref, w2t_ref, b2_ref, w3t_ref, b3_ref,
          pts_ref, lbl_ref, feat_ref, *, width, rows):
    i = pl.program_id(0)
    d = depth_ref[0]
    lab = sem_ref[0]
    bsz = d.shape[1]

    valid = (d > 0.1) & (d < 10.0)
    lbl = jnp.where(valid, lab, 0)
    lbl_ref[0] = lbl

    pidx = lax.broadcasted_iota(jnp.int32, (1, bsz), 1)
    colf = (pidx % width).astype(jnp.float32)
    rowf = (i * rows + pidx // width).astype(jnp.float32)
    cx = params_ref[0]
    cy = params_ref[1]
    rfx = params_ref[2]
    rfy = params_ref[3]
    zero = jnp.zeros_like(d)
    x = jnp.where(valid, (colf - cx) * d * rfx, zero)
    y = jnp.where(valid, (rowf - cy) * d * rfy, zero)
    z = jnp.where(valid, d, zero)
    pts_t = jnp.concatenate([x, y, z], axis=0)
    pts_ref[...] = pts_t

    lc = jnp.clip(lbl, 0, _NUM_SEM)
    sem_iota = lax.broadcasted_iota(jnp.int32, (_NSEM1, bsz), 0)
    one_t = jnp.where(sem_iota == lc, 1.0, 0.0).astype(jnp.bfloat16)

    t1t = lax.dot_general(w1bt_ref[...], embt_ref[...],
                          (((1,), (0,)), ((), ())),
                          precision=lax.Precision.HIGHEST,
                          preferred_element_type=jnp.float32)
    h1 = (_mm(w1at_ref[...], pts_t.astype(jnp.bfloat16))
          + _mm(t1t.astype(jnp.bfloat16), one_t) + b1_ref[...])
    h1 = jnp.maximum(h1, 0.0)
    h2 = jnp.maximum(
        _mm(w2t_ref[...], h1.astype(jnp.bfloat16)) + b2_ref[...], 0.0)
    feat_ref[...] = (_mm(w3t_ref[...], h2.astype(jnp.bfloat16))
                     + b3_ref[...])


def kernel(semantic_map, depth, intrinsics, emb_table, w1, b1, w2, b2, w3, b3):
    h, w = depth.shape
    n = h * w
    rows = _ROWS
    bsz = rows * w
    grid = h // rows
    in_dim = 3 + emb_table.shape[1]

    fx, fy = intrinsics[0, 0], intrinsics[1, 1]
    cx, cy = intrinsics[0, 2], intrinsics[1, 2]
    params = jnp.stack([cx, cy, 1.0 / fx, 1.0 / fy]).astype(jnp.float32)

    depth3 = depth.reshape(grid, 1, bsz)
    sem3 = semantic_map.astype(jnp.int32).reshape(grid, 1, bsz)
    embt = emb_table.T
    w1at = w1[:3, :].T.astype(jnp.bfloat16)
    w1bt = w1[3:, :].T
    w2t = w2.T.astype(jnp.bfloat16)
    w3t = w3.T.astype(jnp.bfloat16)
    b1c = b1.reshape(-1, 1)
    b2c = b2.reshape(-1, 1)
    b3c = b3.reshape(-1, 1)

    body = functools.partial(_body, width=w, rows=rows)

    const = lambda *shape: pl.BlockSpec(shape, lambda i: (0,) * len(shape))
    pts_t, lbl3, feat_t = pl.pallas_call(
        body,
        grid=(grid,),
        in_specs=[
            pl.BlockSpec(memory_space=pltpu.SMEM),
            pl.BlockSpec((1, 1, bsz), lambda i: (i, 0, 0)),
            pl.BlockSpec((1, 1, bsz), lambda i: (i, 0, 0)),
            const(*embt.shape),
            const(*w1at.shape),
            const(*w1bt.shape),
            const(*b1c.shape),
            const(*w2t.shape),
            const(*b2c.shape),
            const(*w3t.shape),
            const(*b3c.shape),
        ],
        out_specs=[
            pl.BlockSpec((3, bsz), lambda i: (0, i)),
            pl.BlockSpec((1, 1, bsz), lambda i: (i, 0, 0)),
            pl.BlockSpec((in_dim, bsz), lambda i: (0, i)),
        ],
        out_shape=[
            jax.ShapeDtypeStruct((3, n), jnp.float32),
            jax.ShapeDtypeStruct((grid, 1, bsz), jnp.int32),
            jax.ShapeDtypeStruct((in_dim, n), jnp.float32),
        ],
        compiler_params=pltpu.CompilerParams(
            dimension_semantics=("arbitrary",)),
    )(params, depth3, sem3, embt, w1at, w1bt, b1c, w2t, b2c, w3t, b3c)

    return pts_t.T, lbl3.reshape(n), feat_t.T
